# Initial kernel scaffold; baseline (speedup 1.0000x reference)
import jax
import jax.numpy as jnp
from jax.experimental import pallas as pl


def kernel(bucket_matrix, bias_table):
    raise NotImplementedError("write your pallas kernel here")


# speedup vs baseline: 20.3645x; 20.3645x over previous
import functools

import jax
import jax.numpy as jnp
from jax import lax
from jax.experimental import pallas as pl
from jax.experimental.pallas import tpu as pltpu
from jax.experimental.pallas import tpu_sc as plsc

NHEAD = 16
NBUCKET = 5
NC = 2
NS = 16
LANES = 16
NW = NC * NS

CB = 2048


def _make_sc_call(n_total):
  assert n_total % (NW * CB) == 0
  chunk = n_total // NW
  nblocks = chunk // CB
  assert nblocks % 2 == 0
  mesh = plsc.VectorSubcoreMesh(core_axis_name="c", subcore_axis_name="s")

  def body(bucket_hbm, table_hbm, out_hbm, table_v,
           bucket0, bucket1, out0, out1,
           in_sem0, in_sem1, out_sem0, out_sem1):
    wid = lax.axis_index("s") * NC + lax.axis_index("c")
    base = wid * chunk
    pltpu.sync_copy(table_hbm, table_v)

    buckets = (bucket0, bucket1)
    outs = (out0, out1)
    in_sems = (in_sem0, in_sem1)
    out_sems = (out_sem0, out_sem1)

    def in_src(blk):
      return bucket_hbm.at[pl.ds(base + blk * CB, CB)]

    def out_dst(blk):
      return out_hbm.at[:, pl.ds(base + blk * CB, CB)]

    for s in range(2):
      pltpu.async_copy(in_src(s), buckets[s], in_sems[s])

    def blk2_body(i0, carry):
      for s in range(2):
        blk = i0 * 2 + s
        bucket_v, out_v = buckets[s], outs[s]
        pltpu.make_async_copy(in_src(blk), bucket_v, in_sems[s]).wait()
        @pl.when(blk >= 2)
        def _():
          pltpu.make_async_copy(out_v, out_dst(blk - 2), out_sems[s]).wait()

        def vec_body(v, carry2):
          off = v * LANES
          bvec = bucket_v[pl.ds(off, LANES)]
          for h in range(NHEAD):
            idx = bvec + (h * NBUCKET)
            out_v[h, pl.ds(off, LANES)] = plsc.load_gather(table_v, [idx])
          return carry2

        lax.fori_loop(0, CB // LANES, vec_body, 0)

        @pl.when(blk + 2 < nblocks)
        def _():
          pltpu.async_copy(in_src(blk + 2), bucket_v, in_sems[s])

        pltpu.async_copy(out_v, out_dst(blk), out_sems[s])
      return carry

    lax.fori_loop(0, nblocks // 2, blk2_body, 0)
    for s in range(2):
      pltpu.make_async_copy(outs[s], out_dst(nblocks - 2 + s), out_sems[s]).wait()

  return pl.kernel(
      body,
      out_type=jax.ShapeDtypeStruct((NHEAD, n_total), jnp.float32),
      mesh=mesh,
      scratch_types=[
          pltpu.VMEM((128,), jnp.float32),
          pltpu.VMEM((CB,), jnp.int32),
          pltpu.VMEM((CB,), jnp.int32),
          pltpu.VMEM((NHEAD, CB), jnp.float32),
          pltpu.VMEM((NHEAD, CB), jnp.float32),
          pltpu.SemaphoreType.DMA,
          pltpu.SemaphoreType.DMA,
          pltpu.SemaphoreType.DMA,
          pltpu.SemaphoreType.DMA,
      ],
      compiler_params=pltpu.CompilerParams(needs_layout_passes=False),
  )


@jax.jit
def kernel(bucket_matrix, bias_table):
  b, l, l2 = bucket_matrix.shape
  n_total = b * l * l2
  flat = bucket_matrix.astype(jnp.int32).reshape(n_total)
  table = jnp.pad(bias_table.astype(jnp.float32).reshape(NHEAD * NBUCKET),
                  (0, 128 - NHEAD * NBUCKET))
  out = _make_sc_call(n_total)(flat, table)
  return out.reshape(NHEAD, b, l, l2).transpose(1, 0, 2, 3)


# speedup vs baseline: 55.3026x; 2.7156x over previous
import functools

import jax
import jax.numpy as jnp
from jax import lax
from jax.experimental import pallas as pl
from jax.experimental.pallas import tpu as pltpu
from jax.experimental.pallas import tpu_sc as plsc

NHEAD = 16
NBUCKET = 5
NC = 2
NS = 16
LANES = 16
NW = NC * NS

HS = 8

BR = 8
BC = 256
ROWS = 64


def _tc_body(bucket_ref, table_ref, out_ref):
  b = bucket_ref[...]
  masks = [b == k for k in range(NBUCKET - 1)]
  for h in range(HS):
    r = jnp.full(b.shape, table_ref[h, NBUCKET - 1], dtype=jnp.float32)
    for k in range(NBUCKET - 2, -1, -1):
      r = jnp.where(masks[k], table_ref[h, k], r)
    out_ref[h] = r


def _make_tc_call(l):
  return pl.pallas_call(
      _tc_body,
      out_shape=jax.ShapeDtypeStruct((HS, l, l), jnp.float32),
      grid=(l // ROWS,),
      in_specs=[
          pl.BlockSpec((ROWS, l), lambda g: (g, 0)),
          pl.BlockSpec(memory_space=pltpu.SMEM),
      ],
      out_specs=pl.BlockSpec((HS, ROWS, l), lambda g: (0, g, 0)),
  )


def _make_sc_call(l):
  nheads = NHEAD - HS
  rows_per_w = l // NW
  nchunks = l // BC
  nblocks = (rows_per_w // BR) * nchunks
  mesh = plsc.VectorSubcoreMesh(core_axis_name="c", subcore_axis_name="s")

  def body(bucket_hbm, table_hbm, out_hbm, table_v,
           bucket0, bucket1, out0, out1,
           in_sem0, in_sem1, out_sem0, out_sem1):
    wid = lax.axis_index("s") * NC + lax.axis_index("c")
    row0 = wid * rows_per_w
    pltpu.sync_copy(table_hbm, table_v)

    buckets = (bucket0, bucket1)
    outs = (out0, out1)
    in_sems = (in_sem0, in_sem1)
    out_sems = (out_sem0, out_sem1)

    def rowslice(blk):
      return pl.ds(row0 + (blk // nchunks) * BR, BR)

    def colslice(blk):
      return pl.ds((blk % nchunks) * BC, BC)

    def in_src(blk):
      return bucket_hbm.at[rowslice(blk), colslice(blk)]

    def out_dst(blk):
      return out_hbm.at[:, rowslice(blk), colslice(blk)]

    for s in range(2):
      pltpu.async_copy(in_src(s), buckets[s], in_sems[s])

    def blk2_body(i0, carry):
      for s in range(2):
        blk = i0 * 2 + s
        bucket_v, out_v = buckets[s], outs[s]
        pltpu.make_async_copy(in_src(blk), bucket_v, in_sems[s]).wait()

        @pl.when(blk >= 2)
        def _():
          pltpu.make_async_copy(out_v, out_dst(blk - 2), out_sems[s]).wait()

        @plsc.parallel_loop(0, BR * BC // LANES, step=1, unroll=4)
        def vec_body(v):
          r = v >> 4
          c = (v & (BC // LANES - 1)) * LANES
          bvec = bucket_v[r, pl.ds(c, LANES)]
          for h in range(HS, NHEAD):
            idx = bvec + (h * NBUCKET)
            out_v[h - HS, r, pl.ds(c, LANES)] = plsc.load_gather(
                table_v, [idx])

        @pl.when(blk + 2 < nblocks)
        def _():
          pltpu.async_copy(in_src(blk + 2), bucket_v, in_sems[s])

        pltpu.async_copy(out_v, out_dst(blk), out_sems[s])
      return carry

    lax.fori_loop(0, nblocks // 2, blk2_body, 0)
    for s in range(2):
      pltpu.make_async_copy(outs[s], out_dst(nblocks - 2 + s), out_sems[s]).wait()

  return pl.kernel(
      body,
      out_type=jax.ShapeDtypeStruct((nheads, l, l), jnp.float32),
      mesh=mesh,
      scratch_types=[
          pltpu.VMEM((128,), jnp.float32),
          pltpu.VMEM((BR, BC), jnp.int32),
          pltpu.VMEM((BR, BC), jnp.int32),
          pltpu.VMEM((nheads, BR, BC), jnp.float32),
          pltpu.VMEM((nheads, BR, BC), jnp.float32),
          pltpu.SemaphoreType.DMA,
          pltpu.SemaphoreType.DMA,
          pltpu.SemaphoreType.DMA,
          pltpu.SemaphoreType.DMA,
      ],
      compiler_params=pltpu.CompilerParams(
          needs_layout_passes=False, use_tc_tiling_on_sc=True),
  )


@jax.jit
def kernel(bucket_matrix, bias_table):
  b, l, l2 = bucket_matrix.shape
  bm = bucket_matrix.astype(jnp.int32).reshape(l, l2)
  table = jnp.pad(bias_table.astype(jnp.float32).reshape(NHEAD * NBUCKET),
                  (0, 128 - NHEAD * NBUCKET))
  out_tc = _make_tc_call(l)(bm, bias_table.astype(jnp.float32))
  out_sc = _make_sc_call(l)(bm, table)
  out = jnp.concatenate([out_tc, out_sc], axis=0)
  return out.reshape(b, NHEAD, l, l2)


# speedup vs baseline: 107.3043x; 1.9403x over previous
import functools

import jax
import jax.numpy as jnp
from jax import lax
from jax.experimental import pallas as pl
from jax.experimental.pallas import tpu as pltpu
from jax.experimental.pallas import tpu_sc as plsc

NHEAD = 16
NBUCKET = 5
NC = 2
NS = 16
LANES = 16
NW = NC * NS

BR = 8
BC = 256


def _make_sc_call(l):
  rows_per_w = l // NW
  nslabs = rows_per_w // BR
  nchunks = l // BC
  nblocks = nslabs * nchunks
  assert nblocks % 2 == 0
  mesh = plsc.VectorSubcoreMesh(core_axis_name="c", subcore_axis_name="s")

  def body(bucket_hbm, table_hbm, out_hbm, table_v,
           bucket0, bucket1, out0, out1,
           in_sem0, in_sem1, out_sem0, out_sem1):
    wid = lax.axis_index("s") * NC + lax.axis_index("c")
    row0 = wid * rows_per_w
    pltpu.sync_copy(table_hbm, table_v)

    buckets = (bucket0, bucket1)
    outs = (out0, out1)
    in_sems = (in_sem0, in_sem1)
    out_sems = (out_sem0, out_sem1)

    def rowslice(blk):
      return pl.ds(row0 + (blk // nchunks) * BR, BR)

    def colslice(blk):
      return pl.ds((blk % nchunks) * BC, BC)

    def in_src(blk):
      return bucket_hbm.at[rowslice(blk), colslice(blk)]

    def out_dst(blk):
      return out_hbm.at[:, rowslice(blk), colslice(blk)]

    for s in range(2):
      pltpu.async_copy(in_src(s), buckets[s], in_sems[s])

    def blk2_body(i0, carry):
      for s in range(2):
        blk = i0 * 2 + s
        bucket_v, out_v = buckets[s], outs[s]
        pltpu.make_async_copy(in_src(blk), bucket_v, in_sems[s]).wait()

        @pl.when(blk >= 2)
        def _():
          pltpu.make_async_copy(out_v, out_dst(blk - 2), out_sems[s]).wait()

        @plsc.parallel_loop(0, BR * BC // LANES, step=1, unroll=8)
        def vec_body(v):
          r = v >> 4
          c = (v & (BC // LANES - 1)) * LANES
          bvec = bucket_v[r, pl.ds(c, LANES)]
          for h in range(NHEAD):
            idx = bvec + (h * NBUCKET)
            out_v[h, r, pl.ds(c, LANES)] = plsc.load_gather(table_v, [idx])

        @pl.when(blk + 2 < nblocks)
        def _():
          pltpu.async_copy(in_src(blk + 2), bucket_v, in_sems[s])

        pltpu.async_copy(out_v, out_dst(blk), out_sems[s])
      return carry

    lax.fori_loop(0, nblocks // 2, blk2_body, 0)
    for s in range(2):
      pltpu.make_async_copy(outs[s], out_dst(nblocks - 2 + s), out_sems[s]).wait()

  return pl.kernel(
      body,
      out_type=jax.ShapeDtypeStruct((NHEAD, l, l), jnp.float32),
      mesh=mesh,
      scratch_types=[
          pltpu.VMEM((128,), jnp.float32),
          pltpu.VMEM((BR, BC), jnp.int32),
          pltpu.VMEM((BR, BC), jnp.int32),
          pltpu.VMEM((NHEAD, BR, BC), jnp.float32),
          pltpu.VMEM((NHEAD, BR, BC), jnp.float32),
          pltpu.SemaphoreType.DMA,
          pltpu.SemaphoreType.DMA,
          pltpu.SemaphoreType.DMA,
          pltpu.SemaphoreType.DMA,
      ],
      compiler_params=pltpu.CompilerParams(
          needs_layout_passes=False, use_tc_tiling_on_sc=True),
  )


@jax.jit
def kernel(bucket_matrix, bias_table):
  b, l, l2 = bucket_matrix.shape
  bm = bucket_matrix.astype(jnp.int32).reshape(l, l2)
  table = jnp.pad(bias_table.astype(jnp.float32).reshape(NHEAD * NBUCKET),
                  (0, 128 - NHEAD * NBUCKET))
  out = _make_sc_call(l)(bm, table)
  return out.reshape(b, NHEAD, l, l2)


# speedup vs baseline: 129.3267x; 1.2052x over previous
import functools

import jax
import jax.numpy as jnp
from jax import lax
from jax.experimental import pallas as pl
from jax.experimental.pallas import tpu as pltpu
from jax.experimental.pallas import tpu_sc as plsc

NHEAD = 16
NBUCKET = 5
NC = 2
NS = 16
LANES = 16
NW = NC * NS

BR = 8
BC = 256


def _make_sc_call(l):
  rows_per_w = l // NW
  nslabs = rows_per_w // BR
  nchunks = l // BC
  nblocks = nslabs * nchunks
  assert nblocks % 2 == 0
  mesh = plsc.VectorSubcoreMesh(core_axis_name="c", subcore_axis_name="s")

  def body(bucket_hbm, table_hbm, out_hbm, table_v,
           bucket0, bucket1, out0, out1,
           in_sem0, in_sem1, out_sem0, out_sem1):
    wid = lax.axis_index("s") * NC + lax.axis_index("c")
    row0 = wid * rows_per_w
    pltpu.sync_copy(table_hbm, table_v)

    buckets = (bucket0, bucket1)
    outs = (out0, out1)
    in_sems = (in_sem0, in_sem1)
    out_sems = (out_sem0, out_sem1)

    def rowslice(blk):
      return pl.ds(row0 + (blk // nchunks) * BR, BR)

    def colslice(blk):
      return pl.ds((blk % nchunks) * BC, BC)

    def in_src(blk):
      return bucket_hbm.at[rowslice(blk), colslice(blk)]

    def out_dst(blk):
      return out_hbm.at[:, rowslice(blk), colslice(blk)]

    for s in range(2):
      pltpu.async_copy(in_src(s), buckets[s], in_sems[s])

    def blk2_body(i0, carry):
      for s in range(2):
        blk = i0 * 2 + s
        bucket_v, out_v = buckets[s], outs[s]
        pltpu.make_async_copy(in_src(blk), bucket_v, in_sems[s]).wait()

        @pl.when(blk >= 2)
        def _():
          pltpu.make_async_copy(out_v, out_dst(blk - 2), out_sems[s]).wait()

        @plsc.parallel_loop(0, BR * BC // LANES, step=1, unroll=2)
        def vec_body(v):
          r = v >> 4
          c = (v & (BC // LANES - 1)) * LANES
          bvec = bucket_v[r, pl.ds(c, LANES)]
          for h in range(NHEAD):
            idx = bvec + (h * NBUCKET)
            out_v[h, r, pl.ds(c, LANES)] = plsc.load_gather(table_v, [idx])

        @pl.when(blk + 2 < nblocks)
        def _():
          pltpu.async_copy(in_src(blk + 2), bucket_v, in_sems[s])

        pltpu.async_copy(out_v, out_dst(blk), out_sems[s])
      return carry

    lax.fori_loop(0, nblocks // 2, blk2_body, 0)
    for s in range(2):
      pltpu.make_async_copy(outs[s], out_dst(nblocks - 2 + s), out_sems[s]).wait()

  return pl.kernel(
      body,
      out_type=jax.ShapeDtypeStruct((NHEAD, l, l), jnp.float32),
      mesh=mesh,
      scratch_types=[
          pltpu.VMEM((128,), jnp.float32),
          pltpu.VMEM((BR, BC), jnp.int32),
          pltpu.VMEM((BR, BC), jnp.int32),
          pltpu.VMEM((NHEAD, BR, BC), jnp.float32),
          pltpu.VMEM((NHEAD, BR, BC), jnp.float32),
          pltpu.SemaphoreType.DMA,
          pltpu.SemaphoreType.DMA,
          pltpu.SemaphoreType.DMA,
          pltpu.SemaphoreType.DMA,
      ],
      compiler_params=pltpu.CompilerParams(
          needs_layout_passes=False, use_tc_tiling_on_sc=True),
  )


@jax.jit
def kernel(bucket_matrix, bias_table):
  b, l, l2 = bucket_matrix.shape
  bm = bucket_matrix.astype(jnp.int32).reshape(l, l2)
  table = jnp.pad(bias_table.astype(jnp.float32).reshape(NHEAD * NBUCKET),
                  (0, 128 - NHEAD * NBUCKET))
  out = _make_sc_call(l)(bm, table)
  return out.reshape(b, NHEAD, l, l2)


# speedup vs baseline: 130.3509x; 1.0079x over previous
import functools

import jax
import jax.numpy as jnp
from jax import lax
from jax.experimental import pallas as pl
from jax.experimental.pallas import tpu as pltpu
from jax.experimental.pallas import tpu_sc as plsc

NHEAD = 16
NBUCKET = 5
NC = 2
NS = 16
LANES = 16
NW = NC * NS

BR = 8
BC = 256


def _make_sc_call(l):
  rows_per_w = l // NW
  nslabs = rows_per_w // BR
  nchunks = l // BC
  nblocks = nslabs * nchunks
  assert nblocks % 2 == 0
  mesh = plsc.VectorSubcoreMesh(core_axis_name="c", subcore_axis_name="s")

  def body(bucket_hbm, table_hbm, out_hbm, table_v,
           bucket0, bucket1, out0, out1,
           in_sem0, in_sem1, out_sem0, out_sem1):
    wid = lax.axis_index("s") * NC + lax.axis_index("c")
    row0 = wid * rows_per_w
    pltpu.sync_copy(table_hbm, table_v)

    buckets = (bucket0, bucket1)
    outs = (out0, out1)
    in_sems = (in_sem0, in_sem1)
    out_sems = (out_sem0, out_sem1)

    def rowslice(blk):
      return pl.ds(row0 + (blk // nchunks) * BR, BR)

    def colslice(blk):
      return pl.ds((blk % nchunks) * BC, BC)

    def in_src(blk):
      return bucket_hbm.at[rowslice(blk), colslice(blk)]

    def out_dst(blk):
      return out_hbm.at[:, rowslice(blk), colslice(blk)]

    for s in range(2):
      pltpu.async_copy(in_src(s), buckets[s], in_sems[s])

    def blk2_body(i0, carry):
      for s in range(2):
        blk = i0 * 2 + s
        bucket_v, out_v = buckets[s], outs[s]
        pltpu.make_async_copy(in_src(blk), bucket_v, in_sems[s]).wait()

        @pl.when(blk >= 2)
        def _():
          pltpu.make_async_copy(out_v, out_dst(blk - 2), out_sems[s]).wait()

        @plsc.parallel_loop(0, BR * BC // LANES, step=1)
        def vec_body(v):
          r = v >> 4
          c = (v & (BC // LANES - 1)) * LANES
          bvec = bucket_v[r, pl.ds(c, LANES)]
          for h in range(NHEAD):
            idx = bvec + (h * NBUCKET)
            out_v[h, r, pl.ds(c, LANES)] = plsc.load_gather(table_v, [idx])

        @pl.when(blk + 2 < nblocks)
        def _():
          pltpu.async_copy(in_src(blk + 2), bucket_v, in_sems[s])

        pltpu.async_copy(out_v, out_dst(blk), out_sems[s])
      return carry

    lax.fori_loop(0, nblocks // 2, blk2_body, 0)
    for s in range(2):
      pltpu.make_async_copy(outs[s], out_dst(nblocks - 2 + s), out_sems[s]).wait()

  return pl.kernel(
      body,
      out_type=jax.ShapeDtypeStruct((NHEAD, l, l), jnp.float32),
      mesh=mesh,
      scratch_types=[
          pltpu.VMEM((128,), jnp.float32),
          pltpu.VMEM((BR, BC), jnp.int32),
          pltpu.VMEM((BR, BC), jnp.int32),
          pltpu.VMEM((NHEAD, BR, BC), jnp.float32),
          pltpu.VMEM((NHEAD, BR, BC), jnp.float32),
          pltpu.SemaphoreType.DMA,
          pltpu.SemaphoreType.DMA,
          pltpu.SemaphoreType.DMA,
          pltpu.SemaphoreType.DMA,
      ],
      compiler_params=pltpu.CompilerParams(
          needs_layout_passes=False, use_tc_tiling_on_sc=True),
  )


@jax.jit
def kernel(bucket_matrix, bias_table):
  b, l, l2 = bucket_matrix.shape
  bm = bucket_matrix.astype(jnp.int32).reshape(l, l2)
  table = jnp.pad(bias_table.astype(jnp.float32).reshape(NHEAD * NBUCKET),
                  (0, 128 - NHEAD * NBUCKET))
  out = _make_sc_call(l)(bm, table)
  return out.reshape(b, NHEAD, l, l2)
